# Initial kernel scaffold; baseline (speedup 1.0000x reference)
import jax
import jax.numpy as jnp
from jax.experimental import pallas as pl


def kernel(h, embeddings):
    raise NotImplementedError("write your pallas kernel here")


# speedup vs baseline: 54.8266x; 54.8266x over previous
import functools

import jax
import jax.numpy as jnp
from jax import lax
from jax.experimental import pallas as pl
from jax.experimental.pallas import tpu as pltpu
from jax.experimental.pallas import tpu_sc as plsc

_K = 512
_N = 2 * 576 * 96
_NC = 2
_NS = 16
_NW = _NC * _NS
_PER = _N // _NW
_L = 16


def _prep_body(er_ref, ec_ref, sv_ref, mi_ref):
    a = jnp.broadcast_to(er_ref[...], (_K, _K))
    b = jnp.broadcast_to(ec_ref[...], (_K, _K))
    ii = lax.broadcasted_iota(jnp.int32, (_K, _K), 0)
    kk = lax.broadcasted_iota(jnp.int32, (_K, _K), 1)
    lt = (a < b).astype(jnp.int32)
    eq_before = ((a == b) & (kk < ii)).astype(jnp.int32)
    rank = jnp.sum(lt + eq_before, axis=1, keepdims=True)
    onehot = rank == kk
    sv = jnp.sum(jnp.where(onehot, b, 0.0), axis=0, keepdims=True)
    eqv = b == jnp.broadcast_to(sv, (_K, _K))
    mi = jnp.max(jnp.where(eqv, ii, -1), axis=0, keepdims=True)
    sv_ref[...] = sv
    mi_ref[...] = mi.astype(jnp.float32)


_prep = pl.pallas_call(
    _prep_body,
    out_shape=(
        jax.ShapeDtypeStruct((1, _K), jnp.float32),
        jax.ShapeDtypeStruct((1, _K), jnp.float32),
    ),
)

def _search_body(h_hbm, sv_hbm, mi_hbm, out_hbm, x_v, o_v, sv_v, mi_v):
    wid = lax.axis_index("s") * _NC + lax.axis_index("c")
    base = wid * _PER
    pltpu.sync_copy(sv_hbm, sv_v)
    pltpu.sync_copy(mi_hbm, mi_v)
    pltpu.sync_copy(h_hbm.at[pl.ds(base, _PER)], x_v)

    def body(i, carry):
        x = x_v[pl.ds(i * _L, _L)]
        j = jnp.zeros((_L,), jnp.int32)
        step = _K // 2
        while step >= 1:
            probe = j + (step - 1)
            v = plsc.load_gather(sv_v, [probe])
            j = jnp.where(v < x, j + step, j)
            step //= 2
        lo = jnp.maximum(j - 1, 0)
        vlo = plsc.load_gather(sv_v, [lo])
        vhi = plsc.load_gather(sv_v, [j])
        milo = plsc.load_gather(mi_v, [lo])
        mihi = plsc.load_gather(mi_v, [j])
        dlo = jnp.abs(x - vlo)
        dhi = jnp.abs(vhi - x)
        pick_hi = (dhi < dlo) | ((dhi == dlo) & (mihi > milo))
        o_v[pl.ds(i * _L, _L)] = jnp.where(pick_hi, vhi, vlo)
        return carry

    lax.fori_loop(0, _PER // _L, body, 0)
    pltpu.sync_copy(o_v, out_hbm.at[pl.ds(base, _PER)])


@functools.cache
def _make_search():
    mesh = plsc.VectorSubcoreMesh(
        core_axis_name="c", subcore_axis_name="s", num_cores=_NC, num_subcores=_NS
    )
    return pl.kernel(
        _search_body,
        out_type=jax.ShapeDtypeStruct((_N,), jnp.float32),
        mesh=mesh,
        scratch_types=[
            pltpu.VMEM((_PER,), jnp.float32),
            pltpu.VMEM((_PER,), jnp.float32),
            pltpu.VMEM((_K,), jnp.float32),
            pltpu.VMEM((_K,), jnp.float32),
        ],
        compiler_params=pltpu.CompilerParams(needs_layout_passes=False),
    )


def kernel(h, embeddings):
    sv, mi = _prep(embeddings.reshape(1, _K), embeddings.reshape(_K, 1))
    q = _make_search()(h.reshape(_N), sv.reshape(_K), mi.reshape(_K))
    return q.reshape(h.shape)


# speedup vs baseline: 75.0384x; 1.3686x over previous
import functools

import jax
import jax.numpy as jnp
from jax import lax
from jax.experimental import pallas as pl
from jax.experimental.pallas import tpu as pltpu
from jax.experimental.pallas import tpu_sc as plsc

_K = 512
_N = 2 * 576 * 96
_NC = 2
_NS = 16
_NW = _NC * _NS
_PER = _N // _NW
_L = 16


def _prep_body(er_ref, ec_ref, sv_ref, mi_ref):
    a = jnp.broadcast_to(er_ref[...], (_K, _K))
    b = jnp.broadcast_to(ec_ref[...], (_K, _K))
    ii = lax.broadcasted_iota(jnp.int32, (_K, _K), 0)
    kk = lax.broadcasted_iota(jnp.int32, (_K, _K), 1)
    lt = (a < b).astype(jnp.int32)
    eq_before = ((a == b) & (kk < ii)).astype(jnp.int32)
    rank = jnp.sum(lt + eq_before, axis=1, keepdims=True)
    onehot = rank == kk
    sv = jnp.sum(jnp.where(onehot, b, 0.0), axis=0, keepdims=True)
    eqv = b == jnp.broadcast_to(sv, (_K, _K))
    mi = jnp.max(jnp.where(eqv, ii, -1), axis=0, keepdims=True)
    sv_ref[...] = sv
    mi_ref[...] = mi.astype(jnp.float32)


_prep = pl.pallas_call(
    _prep_body,
    out_shape=(
        jax.ShapeDtypeStruct((1, _K), jnp.float32),
        jax.ShapeDtypeStruct((1, _K), jnp.float32),
    ),
)

def _search_body(h_hbm, sv_hbm, mi_hbm, out_hbm, x_v, o_v, sv_v, mi_v):
    wid = lax.axis_index("s") * _NC + lax.axis_index("c")
    base = wid * _PER
    pltpu.sync_copy(sv_hbm, sv_v)
    pltpu.sync_copy(mi_hbm, mi_v)
    pltpu.sync_copy(h_hbm.at[pl.ds(base, _PER)], x_v)

    @plsc.parallel_loop(0, _PER // _L, unroll=8)
    def body(i):
        x = x_v[pl.ds(i * _L, _L)]
        j = jnp.zeros((_L,), jnp.int32)
        step = _K // 2
        while step >= 1:
            probe = j + (step - 1)
            v = plsc.load_gather(sv_v, [probe])
            j = jnp.where(v < x, j + step, j)
            step //= 2
        lo = jnp.maximum(j - 1, 0)
        vlo = plsc.load_gather(sv_v, [lo])
        vhi = plsc.load_gather(sv_v, [j])
        milo = plsc.load_gather(mi_v, [lo])
        mihi = plsc.load_gather(mi_v, [j])
        dlo = jnp.abs(x - vlo)
        dhi = jnp.abs(vhi - x)
        pick_hi = (dhi < dlo) | ((dhi == dlo) & (mihi > milo))
        o_v[pl.ds(i * _L, _L)] = jnp.where(pick_hi, vhi, vlo)

    pltpu.sync_copy(o_v, out_hbm.at[pl.ds(base, _PER)])


@functools.cache
def _make_search():
    mesh = plsc.VectorSubcoreMesh(
        core_axis_name="c", subcore_axis_name="s", num_cores=_NC, num_subcores=_NS
    )
    return pl.kernel(
        _search_body,
        out_type=jax.ShapeDtypeStruct((_N,), jnp.float32),
        mesh=mesh,
        scratch_types=[
            pltpu.VMEM((_PER,), jnp.float32),
            pltpu.VMEM((_PER,), jnp.float32),
            pltpu.VMEM((_K,), jnp.float32),
            pltpu.VMEM((_K,), jnp.float32),
        ],
        compiler_params=pltpu.CompilerParams(needs_layout_passes=False),
    )


def kernel(h, embeddings):
    sv, mi = _prep(embeddings.reshape(1, _K), embeddings.reshape(_K, 1))
    q = _make_search()(h.reshape(_N), sv.reshape(_K), mi.reshape(_K))
    return q.reshape(h.shape)


# speedup vs baseline: 77.9208x; 1.0384x over previous
import functools

import jax
import jax.numpy as jnp
from jax import lax
from jax.experimental import pallas as pl
from jax.experimental.pallas import tpu as pltpu
from jax.experimental.pallas import tpu_sc as plsc

_K = 512
_N = 2 * 576 * 96
_NC = 2
_NS = 16
_NW = _NC * _NS
_PER = _N // _NW
_L = 16


def _prep_body(er_ref, ec_ref, sv_ref, mi_ref):
    a = jnp.broadcast_to(er_ref[...], (_K, _K))
    b = jnp.broadcast_to(ec_ref[...], (_K, _K))
    ii = lax.broadcasted_iota(jnp.int32, (_K, _K), 0)
    kk = lax.broadcasted_iota(jnp.int32, (_K, _K), 1)
    lt = (a < b).astype(jnp.int32)
    eq_before = ((a == b) & (kk < ii)).astype(jnp.int32)
    rank = jnp.sum(lt + eq_before, axis=1, keepdims=True)
    onehot = rank == kk
    sv = jnp.sum(jnp.where(onehot, b, 0.0), axis=0, keepdims=True)
    eqv = b == jnp.broadcast_to(sv, (_K, _K))
    mi = jnp.max(jnp.where(eqv, ii, -1), axis=0, keepdims=True)
    sv_ref[...] = sv
    mi_ref[...] = mi.astype(jnp.float32)


_prep = pl.pallas_call(
    _prep_body,
    out_shape=(
        jax.ShapeDtypeStruct((1, _K), jnp.float32),
        jax.ShapeDtypeStruct((1, _K), jnp.float32),
    ),
)

def _search_body(h_hbm, sv_hbm, mi_hbm, out_hbm, x_v, o_v, sv_v, mi_v):
    wid = lax.axis_index("s") * _NC + lax.axis_index("c")
    base = wid * _PER
    pltpu.sync_copy(sv_hbm, sv_v)
    pltpu.sync_copy(mi_hbm, mi_v)
    pltpu.sync_copy(h_hbm.at[pl.ds(base, _PER)], x_v)

    @plsc.parallel_loop(0, _PER // _L, unroll=24)
    def body(i):
        x = x_v[pl.ds(i * _L, _L)]
        j = jnp.zeros((_L,), jnp.int32)
        step = _K // 2
        while step >= 1:
            probe = j + (step - 1)
            v = plsc.load_gather(sv_v, [probe])
            j = jnp.where(v < x, j + step, j)
            step //= 2
        lo = jnp.maximum(j - 1, 0)
        vlo = plsc.load_gather(sv_v, [lo])
        vhi = plsc.load_gather(sv_v, [j])
        milo = plsc.load_gather(mi_v, [lo])
        mihi = plsc.load_gather(mi_v, [j])
        dlo = jnp.abs(x - vlo)
        dhi = jnp.abs(vhi - x)
        pick_hi = (dhi < dlo) | ((dhi == dlo) & (mihi > milo))
        o_v[pl.ds(i * _L, _L)] = jnp.where(pick_hi, vhi, vlo)

    pltpu.sync_copy(o_v, out_hbm.at[pl.ds(base, _PER)])


@functools.cache
def _make_search():
    mesh = plsc.VectorSubcoreMesh(
        core_axis_name="c", subcore_axis_name="s", num_cores=_NC, num_subcores=_NS
    )
    return pl.kernel(
        _search_body,
        out_type=jax.ShapeDtypeStruct((_N,), jnp.float32),
        mesh=mesh,
        scratch_types=[
            pltpu.VMEM((_PER,), jnp.float32),
            pltpu.VMEM((_PER,), jnp.float32),
            pltpu.VMEM((_K,), jnp.float32),
            pltpu.VMEM((_K,), jnp.float32),
        ],
        compiler_params=pltpu.CompilerParams(needs_layout_passes=False),
    )


def kernel(h, embeddings):
    sv, mi = _prep(embeddings.reshape(1, _K), embeddings.reshape(_K, 1))
    q = _make_search()(h.reshape(_N), sv.reshape(_K), mi.reshape(_K))
    return q.reshape(h.shape)
